# baseline (speedup 1.0000x reference)
import jax
import jax.numpy as jnp
from jax import lax
from jax.experimental import pallas as pl
from jax.experimental.pallas import tpu as pltpu
from jax.experimental.pallas import tpu_sc as plsc

_N = 1024
_K = 1024
_D = 64
_L = 16
_RPW = 32
_HALF = 16
_NT = 4
_KT = 8
_KV = _K // _L
_KC = 512


def _all_lanes_reduce(v, op):
    idx = lax.iota(jnp.int32, _L)
    dnums = lax.GatherDimensionNumbers(
        offset_dims=(), collapsed_slice_dims=(0,), start_index_map=(0,)
    )
    for sh in (8, 4, 2, 1):
        perm = jnp.bitwise_and(idx + sh, _L - 1)
        rot = lax.gather(
            v, perm[:, None], dnums, slice_sizes=(1,),
            mode=lax.GatherScatterMode.PROMISE_IN_BOUNDS,
        )
        v = op(v, rot)
    return v


def _sc_body(xsp_hbm, ct_hbm, o_hbm, xsp_v, ct_v, logit_v):
    c = lax.axis_index("c")
    s = lax.axis_index("s")
    w = s * 2 + c
    row0 = w * _RPW

    def half_body(half, carry0):
        rbase = row0 + half * _HALF
        pltpu.sync_copy(xsp_hbm.at[pl.ds(rbase, _HALF)], xsp_v)

        def kc_body(kc, carry1):
            pltpu.sync_copy(ct_hbm.at[kc], ct_v)
            kv0 = kc * (_KC // _L)

            def tile_body(t, carry2):
                nb = (t // (_KC // _L // _KT)) * _NT
                kb = (t % (_KC // _L // _KT)) * _KT

                def d_body(d, accs):
                    cts = [ct_v[d, pl.ds((kb + j) * _L, _L)] for j in range(_KT)]
                    new = []
                    for i in range(_NT):
                        xv = xsp_v[nb + i, pl.ds(d * _L, _L)]
                        for j in range(_KT):
                            new.append(accs[i * _KT + j] + jnp.abs(xv - cts[j]))
                    return tuple(new)

                init = tuple(
                    jnp.zeros((_L,), jnp.float32) for _ in range(_NT * _KT)
                )
                accs = lax.fori_loop(0, _D, d_body, init)
                for i in range(_NT):
                    for j in range(_KT):
                        a = accs[i * _KT + j]
                        logit_v[nb + i, pl.ds((kv0 + kb + j) * _L, _L)] = (
                            a * a * (-0.5)
                        )
                return carry2

            lax.fori_loop(0, (_HALF // _NT) * (_KC // _L // _KT), tile_body, 0)
            return carry1

        lax.fori_loop(0, _K // _KC, kc_body, 0)

        def srow(i, carry1):
            def mx(kv, m):
                return jnp.maximum(m, logit_v[i, pl.ds(kv * _L, _L)])

            m = lax.fori_loop(0, _KV, mx, jnp.full((_L,), -1e30, jnp.float32))
            ms = _all_lanes_reduce(m, jnp.maximum)

            def ex(kv, acc):
                e = jnp.exp(logit_v[i, pl.ds(kv * _L, _L)] - ms)
                logit_v[i, pl.ds(kv * _L, _L)] = e
                return acc + e

            sv = lax.fori_loop(0, _KV, ex, jnp.zeros((_L,), jnp.float32))
            inv = 1.0 / _all_lanes_reduce(sv, jnp.add)

            def dv(kv, carry2):
                logit_v[i, pl.ds(kv * _L, _L)] = logit_v[i, pl.ds(kv * _L, _L)] * inv
                return carry2

            lax.fori_loop(0, _KV, dv, 0)
            return carry1

        lax.fori_loop(0, _HALF, srow, 0)
        pltpu.sync_copy(logit_v, o_hbm.at[pl.ds(rbase, _HALF)])
        return carry0

    lax.fori_loop(0, _RPW // _HALF, half_body, 0)


def kernel(x, placeCells):
    x = jnp.reshape(x, (-1, _D))
    ct = placeCells.T
    ct4 = jnp.transpose(
        jnp.reshape(ct, (_D, _K // _KC, _KC)), (1, 0, 2)
    )
    xsp = jnp.reshape(
        jnp.broadcast_to(x[:, :, None], (_N, _D, _L)), (_N, _D * _L)
    )
    mesh = plsc.VectorSubcoreMesh(core_axis_name="c", subcore_axis_name="s")
    f = pl.kernel(
        _sc_body,
        out_type=jax.ShapeDtypeStruct((_N, _K), jnp.float32),
        mesh=mesh,
        scratch_types=[
            pltpu.VMEM((_HALF, _D * _L), jnp.float32),
            pltpu.VMEM((_D, _KC), jnp.float32),
            pltpu.VMEM((_HALF, _K), jnp.float32),
        ],
    )
    return f(xsp, ct4)
